# Initial kernel scaffold; baseline (speedup 1.0000x reference)
import jax
import jax.numpy as jnp
from jax.experimental import pallas as pl


def kernel(x, edge_index, batch, W1_0, b1_0, W2_0, b2_0, W1_1, b1_1, W2_1, b2_1, W1_2, b1_2, W2_2, b2_2):
    raise NotImplementedError("write your pallas kernel here")


# speedup vs baseline: 3.2071x; 3.2071x over previous
import functools

import jax
import jax.numpy as jnp
from jax import lax
from jax.experimental import pallas as pl
from jax.experimental.pallas import tpu as pltpu
from jax.experimental.pallas import tpu_sc as plsc

N = 10000
E = 160000
D = 256
H = 128
G = 128
NS = 16
NC = 2

CHUNK = 128
NBUF = 2
IDEPTH = 8
CH = 80
EDGES_PER_TILE = CH * CHUNK
E_PAD = NS * EDGES_PER_TILE
NGROUPS = CH // IDEPTH
NROWS = N + 8

RPT = 624
RPT_LAST = 640
ROW0_LAST = 15 * RPT

PCH = 5
FULL_CHUNKS = N // CHUNK
TAIL = N - FULL_CHUNKS * CHUNK
GROWS = G + 8
GPT = G // NS

_mesh = plsc.VectorSubcoreMesh(core_axis_name="c", subcore_axis_name="s")


@functools.partial(
    pl.kernel,
    out_type=jax.ShapeDtypeStruct((NC, N, H), jnp.float32),
    mesh=_mesh,
    scratch_types=[
        pltpu.VMEM((IDEPTH, CHUNK), jnp.int32),
        pltpu.VMEM((IDEPTH, CHUNK), jnp.int32),
        pltpu.VMEM((NBUF, CHUNK, H), jnp.float32),
        pltpu.VMEM_SHARED((NROWS, H), jnp.float32),
        pltpu.SemaphoreType.DMA((NBUF,)),
        pltpu.SemaphoreType.DMA((NBUF,)),
        pltpu.SemaphoreType.DMA((IDEPTH,)),
    ],
)
def _sc_message(h2, src_f, dst_f, pre2, srcv, dstv, rows, acc,
                gsem, ssem, isem):
    c = lax.axis_index("c")
    s = lax.axis_index("s")
    table = h2.at[c]
    tile_base = s * EDGES_PER_TILE

    @pl.when(s < 15)
    def _():
        row0 = pl.multiple_of(s * RPT, 8)
        pltpu.sync_copy(table.at[pl.ds(row0, RPT)],
                        acc.at[pl.ds(row0, RPT)])

    @pl.when(s == 15)
    def _():
        pltpu.sync_copy(table.at[pl.ds(ROW0_LAST, RPT_LAST)],
                        acc.at[pl.ds(ROW0_LAST, RPT_LAST)])

    def fire_idx(i, b):
        off = pl.multiple_of(tile_base + i * CHUNK, 8)
        pltpu.async_copy(src_f.at[pl.ds(off, CHUNK)], srcv.at[b], isem.at[b])
        pltpu.async_copy(dst_f.at[pl.ds(off, CHUNK)], dstv.at[b], isem.at[b])

    def wait_idx(b):
        pltpu.make_async_copy(src_f.at[pl.ds(0, CHUNK)], srcv.at[b],
                              isem.at[b]).wait()
        pltpu.make_async_copy(dst_f.at[pl.ds(0, CHUNK)], dstv.at[b],
                              isem.at[b]).wait()

    def fire_gather(bi, br):
        pltpu.async_copy(table.at[srcv.at[bi]], rows.at[br], gsem.at[br])

    def wait_gather(bi, br):
        pltpu.make_async_copy(table.at[srcv.at[bi]], rows.at[br],
                              gsem.at[br]).wait()

    def fire_scatter(bi, br):
        pltpu.async_copy(rows.at[br], acc.at[dstv.at[bi]], ssem.at[br],
                         add=True)

    def wait_scatter(bi, br):
        pltpu.make_async_copy(rows.at[br], acc.at[dstv.at[bi]],
                              ssem.at[br]).wait()

    for j in range(IDEPTH):
        fire_idx(j, j)
    plsc.subcore_barrier()
    for j in range(NBUF):
        wait_idx(j)
        fire_gather(j, j)

    @pl.loop(0, NGROUPS)
    def _(g):
        for b in range(IDEPTH):
            i = g * IDEPTH + b
            br = b % NBUF
            wait_gather(b, br)
            fire_scatter(b, br)
            wait_scatter(b, br)

            @pl.when(i + IDEPTH < CH)
            def _():
                fire_idx(i + IDEPTH, b)

            @pl.when(i + NBUF < CH)
            def _():
                b2 = (b + NBUF) % IDEPTH
                wait_idx(b2)
                fire_gather(b2, br)

    plsc.subcore_barrier()

    @pl.when(s < 15)
    def _():
        row0 = pl.multiple_of(s * RPT, 8)
        pltpu.sync_copy(acc.at[pl.ds(row0, RPT)],
                        pre2.at[c, pl.ds(row0, RPT)])

    @pl.when(s == 15)
    def _():
        pltpu.sync_copy(acc.at[pl.ds(ROW0_LAST, RPT_LAST)],
                        pre2.at[c, pl.ds(ROW0_LAST, RPT_LAST)])


@functools.partial(
    pl.kernel,
    out_type=jax.ShapeDtypeStruct((NC, G, H), jnp.float32),
    mesh=_mesh,
    scratch_types=[
        pltpu.VMEM((PCH, CHUNK), jnp.int32),
        pltpu.VMEM((CHUNK, H), jnp.float32),
        pltpu.VMEM((GPT, H), jnp.float32),
        pltpu.VMEM_SHARED((GROWS, H), jnp.float32),
    ],
)
def _sc_pool(h2, batch_r, pooled, bidx, pbuf, zbuf, acc):
    c = lax.axis_index("c")
    s = lax.axis_index("s")

    @pl.loop(0, GPT)
    def _(r):
        for j in range(H // 16):
            zbuf[r, pl.ds(j * 16, 16)] = jnp.zeros((16,), jnp.float32)
    pltpu.sync_copy(zbuf, acc.at[pl.ds(s * GPT, GPT)])
    @pl.when(s == 0)
    def _():
        pltpu.sync_copy(zbuf.at[pl.ds(0, GROWS - G)],
                        acc.at[pl.ds(G, GROWS - G)])
    pltpu.sync_copy(batch_r.at[s], bidx)
    plsc.subcore_barrier()

    for k in range(PCH):
        ci = s * PCH + k

        @pl.when(ci < FULL_CHUNKS)
        def _():
            off = pl.multiple_of(ci * CHUNK, 8)
            pltpu.sync_copy(h2.at[c, pl.ds(off, CHUNK)], pbuf)

        @pl.when(ci == FULL_CHUNKS)
        def _():
            pltpu.sync_copy(h2.at[c, pl.ds(FULL_CHUNKS * CHUNK, TAIL)],
                            pbuf.at[pl.ds(0, TAIL)])

        pltpu.sync_copy(pbuf, acc.at[bidx.at[k]], add=True)

    plsc.subcore_barrier()
    pltpu.sync_copy(acc.at[pl.ds(s * GPT, GPT)],
                    pooled.at[c, pl.ds(s * GPT, GPT)])


BN = 1000
NB = N // BN


def _tc_mlp_body(pre_ref, w1_ref, b1_ref, w2_ref, b2_ref, out_ref):
    x = jnp.concatenate([pre_ref[0], pre_ref[1]], axis=1)
    t = jnp.maximum(
        jnp.dot(x, w1_ref[...], preferred_element_type=jnp.float32)
        + b1_ref[...], 0.0)
    y = jnp.maximum(
        jnp.dot(t, w2_ref[...], preferred_element_type=jnp.float32)
        + b2_ref[...], 0.0)
    out_ref[0] = y[:, :H]
    out_ref[1] = y[:, H:]


def _tc_mlp(pre2, w1, b1, w2, b2):
    return pl.pallas_call(
        _tc_mlp_body,
        grid=(NB,),
        in_specs=[
            pl.BlockSpec((NC, BN, H), lambda i: (0, i, 0)),
            pl.BlockSpec((D, D), lambda i: (0, 0)),
            pl.BlockSpec((1, D), lambda i: (0, 0)),
            pl.BlockSpec((D, D), lambda i: (0, 0)),
            pl.BlockSpec((1, D), lambda i: (0, 0)),
        ],
        out_specs=pl.BlockSpec((NC, BN, H), lambda i: (0, i, 0)),
        out_shape=jax.ShapeDtypeStruct((NC, N, H), jnp.float32),
    )(pre2, w1, b1, w2, b2)


def _tc_finish_body(pooled_ref, batch_ref, out_ref):
    b = batch_ref[...]
    gi = lax.broadcasted_iota(jnp.int32, (1, G), 1)
    cnt = jnp.sum((b == gi).astype(jnp.float32), axis=0)
    denom = jnp.maximum(cnt, 1.0)[:, None]
    hg = jnp.concatenate([pooled_ref[0], pooled_ref[1]], axis=1)
    out_ref[...] = hg / denom


def _tc_finish(pooled, batch_2d):
    return pl.pallas_call(
        _tc_finish_body,
        out_shape=jax.ShapeDtypeStruct((G, D), jnp.float32),
    )(pooled, batch_2d)


def kernel(x, edge_index, batch, W1_0, b1_0, W2_0, b2_0, W1_1, b1_1, W2_1,
           b2_1, W1_2, b1_2, W2_2, b2_2):
    src = edge_index[0]
    dst = edge_index[1]
    pad = E_PAD - E
    src_f = jnp.concatenate([src, jnp.zeros((pad,), jnp.int32)])
    dst_f = jnp.concatenate([dst, jnp.full((pad,), N, jnp.int32)])

    batch_r = jnp.concatenate(
        [batch, jnp.full((NS * PCH * CHUNK - N,), G, jnp.int32)]
    ).reshape(NS, PCH, CHUNK)
    batch_2d = jnp.concatenate(
        [batch, jnp.full((80 * 128 - N,), -1, jnp.int32)]).reshape(80 * 128, 1)

    h2 = jnp.stack([x[:, :H], x[:, H:]])
    weights = [(W1_0, b1_0, W2_0, b2_0), (W1_1, b1_1, W2_1, b2_1),
               (W1_2, b1_2, W2_2, b2_2)]

    for (w1, b1, w2, b2) in weights:
        pre2 = _sc_message(h2, src_f, dst_f)
        h2 = _tc_mlp(pre2, w1, b1.reshape(1, D), w2, b2.reshape(1, D))

    pooled = _sc_pool(h2, batch_r)
    return _tc_finish(pooled, batch_2d)
